# Initial kernel scaffold; baseline (speedup 1.0000x reference)
import jax
import jax.numpy as jnp
from jax.experimental import pallas as pl


def kernel(seq1hot, idx, node, edge, x, edge_index, params):
    raise NotImplementedError("write your pallas kernel here")


# speedup vs baseline: 27.5332x; 27.5332x over previous
import functools

import jax
import jax.numpy as jnp
import numpy as np
from jax import lax
from jax.experimental import pallas as pl
from jax.experimental.pallas import tpu as pltpu
from jax.experimental.pallas import tpu_sc as plsc

F32 = jnp.float32
H = 4
C = 64
HC = H * C
EPS = 1e-5


def _ln(x, g, be):
    mu = jnp.mean(x, -1, keepdims=True)
    var = jnp.mean((x - mu) * (x - mu), -1, keepdims=True)
    return g * (x - mu) / jnp.sqrt(var + EPS) + be


def _node_embed_body(node_ref, seq_ref, ng, nb, wn, ws, b0, g2, b2, out_ref):
    xn = _ln(node_ref[...], ng[...], nb[...])
    y = jnp.dot(xn, wn[...], preferred_element_type=F32)
    y = y + jnp.dot(seq_ref[...], ws[...], preferred_element_type=F32) + b0[...]
    out_ref[...] = _ln(y, g2[...], b2[...])


def _node_embed(node2, seq2, p):
    n = node2.shape[0]
    full = lambda a: pl.BlockSpec(a.shape, lambda: tuple(0 for _ in a.shape))
    args = (node2, seq2,
            p['norm_node']['g'].reshape(1, -1), p['norm_node']['be'].reshape(1, -1),
            p['embed_x_lin']['W'][:node2.shape[1]],
            p['embed_x_lin']['W'][node2.shape[1]:],
            p['embed_x_lin']['b'].reshape(1, -1),
            p['embed_x_ln']['g'].reshape(1, -1), p['embed_x_ln']['be'].reshape(1, -1))
    return pl.pallas_call(
        _node_embed_body,
        out_shape=jax.ShapeDtypeStruct((n, C), F32),
        in_specs=[full(a) for a in args],
        out_specs=pl.BlockSpec((n, C), lambda: (0, 0)),
    )(*args)


def _edge_embed_body(edge_ref, eg, eb, we1, wss, wbn, b0, g2, b2, out_ref):
    it = edge_ref.shape[1]
    l = edge_ref.shape[2]
    e = edge_ref[0]
    en = _ln(e, eg[...], eb[...])
    proj = jnp.dot(en.reshape(it * l, -1), we1[...], preferred_element_type=F32)
    proj = proj.reshape(it, l, -1)
    i_glob = pl.program_id(1) * it + lax.broadcasted_iota(jnp.int32, (it, l), 0)
    j_glob = lax.broadcasted_iota(jnp.int32, (it, l), 1)
    s = (j_glob - i_glob).astype(F32)
    sign = jnp.sign(s)
    ss = sign * jnp.clip(jnp.log(jnp.abs(s) + 1.0), 0.0, 5.5)
    bn = jnp.where(jnp.abs(s) > 1.0, 0.0, s)
    y = proj + ss[..., None] * wss[...] + bn[..., None] * wbn[...] + b0[...]
    out_ref[0] = _ln(y, g2[...], b2[...])


def _edge_embed(edge, p):
    b, l = edge.shape[0], edge.shape[1]
    ein = edge.shape[3]
    it = 32
    w = p['embed_e_lin']['W']
    args = (p['norm_edge']['g'].reshape(1, 1, -1), p['norm_edge']['be'].reshape(1, 1, -1),
            w[:ein],
            w[ein].reshape(1, 1, -1), w[ein + 1].reshape(1, 1, -1),
            p['embed_e_lin']['b'].reshape(1, 1, -1),
            p['embed_e_ln']['g'].reshape(1, 1, -1), p['embed_e_ln']['be'].reshape(1, 1, -1))
    full = lambda a: pl.BlockSpec(a.shape, lambda bi, ii: tuple(0 for _ in a.shape))
    return pl.pallas_call(
        _edge_embed_body,
        grid=(b, l // it),
        out_shape=jax.ShapeDtypeStruct((b, l, l, C), F32),
        in_specs=[pl.BlockSpec((1, it, l, ein), lambda bi, ii: (bi, ii, 0, 0))]
                 + [full(a) for a in args],
        out_specs=pl.BlockSpec((1, it, l, C), lambda bi, ii: (bi, ii, 0, 0)),
    )(edge, *args)


def _block_body(gxf_ref, gxj_ref, ee_ref, wq, bq, wk, bk, wv, bv, we,
                wskip, bskip, lng, lnb, wlin, blin, out_ref):
    jt = gxj_ref.shape[1]
    l = gxf_ref.shape[1]
    gxf = gxf_ref[0]
    gxj = gxj_ref[0]
    ee = ee_ref[0]
    q = jnp.dot(gxj, wq[...], preferred_element_type=F32) + bq[...]
    k = jnp.dot(gxf, wk[...], preferred_element_type=F32) + bk[...]
    v = jnp.dot(gxf, wv[...], preferred_element_type=F32) + bv[...]
    i_ids = lax.broadcasted_iota(jnp.int32, (l, jt), 0)
    j_ids = pl.program_id(1) * jt + lax.broadcasted_iota(jnp.int32, (l, jt), 1)
    outs = []
    for h in range(H):
        sl = slice(h * C, (h + 1) * C)
        qh = q[:, sl]
        kh = k[:, sl]
        vh = v[:, sl]
        weh = we[:, sl]
        qk = lax.dot_general(kh, qh, (((1,), (1,)), ((), ())),
                             preferred_element_type=F32)
        qe = lax.dot_general(qh, weh, (((1,), (1,)), ((), ())),
                             preferred_element_type=F32)
        ae_log = jnp.sum(ee * qe[None], axis=-1)
        logit = (qk + ae_log) * (1.0 / np.sqrt(C))
        logit = jnp.where(i_ids == j_ids, -1e30, logit)
        m = jnp.max(logit, axis=0, keepdims=True)
        ex = jnp.exp(logit - m)
        den = jnp.sum(ex, axis=0, keepdims=True)
        al = ex / (den + 1e-16)
        outv = lax.dot_general(al, vh, (((0,), (0,)), ((), ())),
                               preferred_element_type=F32)
        ae = jnp.sum(ee * al[:, :, None], axis=0)
        oute = jnp.dot(ae, weh, preferred_element_type=F32)
        outs.append(outv + oute)
    out = jnp.concatenate(outs, axis=1)
    out = out + jnp.dot(gxj, wskip[...], preferred_element_type=F32) + bskip[...]
    out = _ln(out, lng[...], lnb[...])
    z = jnp.dot(out, wlin[...], preferred_element_type=F32) + blin[...] + gxj
    out_ref[0] = jnp.where(z > 0, z, jnp.exp(z) - 1.0)


def _unimp_block(gx, edge_e, bp):
    b, l = gx.shape[0], gx.shape[1]
    jt = 128
    args = (bp['q']['W'], bp['q']['b'].reshape(1, -1),
            bp['k']['W'], bp['k']['b'].reshape(1, -1),
            bp['v']['W'], bp['v']['b'].reshape(1, -1),
            bp['e']['W'],
            bp['skip']['W'], bp['skip']['b'].reshape(1, -1),
            bp['ln']['g'].reshape(1, -1), bp['ln']['be'].reshape(1, -1),
            bp['lin']['W'], bp['lin']['b'].reshape(1, -1))
    full = lambda a: pl.BlockSpec(a.shape, lambda bi, ji: tuple(0 for _ in a.shape))
    return pl.pallas_call(
        _block_body,
        grid=(b, l // jt),
        out_shape=jax.ShapeDtypeStruct((b, l, C), F32),
        in_specs=[pl.BlockSpec((1, l, C), lambda bi, ji: (bi, 0, 0)),
                  pl.BlockSpec((1, jt, C), lambda bi, ji: (bi, ji, 0)),
                  pl.BlockSpec((1, l, jt, C), lambda bi, ji: (bi, 0, ji, 0))]
                 + [full(a) for a in args],
        out_specs=pl.BlockSpec((1, jt, C), lambda bi, ji: (bi, ji, 0)),
    )(gx, gx, edge_e, *args)


def _sc_segsum(x_tab, src2, dst2, zeros_tab, n_iter, kch):
    r = x_tab.shape[0]
    rpt = r // 16
    mesh = plsc.VectorSubcoreMesh(core_axis_name="c", subcore_axis_name="s")

    @functools.partial(
        pl.kernel,
        out_type=jax.ShapeDtypeStruct((2 * r, 64), F32),
        mesh=mesh,
        compiler_params=pltpu.CompilerParams(use_tc_tiling_on_sc=False),
        scratch_types=[
            pltpu.VMEM((kch, 128), jnp.int32),
            pltpu.VMEM((kch, 128), jnp.int32),
            pltpu.VMEM((kch, 128, 64), F32),
            pltpu.VMEM_SHARED((r, 64), F32),
            pltpu.SemaphoreType.DMA,
            pltpu.SemaphoreType.DMA,
        ],
    )
    def seg(x_hbm, src_hbm, dst_hbm, zero_hbm, out_hbm,
            srcv, dstv, rows, aggsh, gsem, ssem):
        ci = lax.axis_index("c")
        si = lax.axis_index("s")
        wid = ci * 16 + si
        pltpu.sync_copy(zero_hbm.at[pl.ds(si * rpt, rpt)],
                        aggsh.at[pl.ds(si * rpt, rpt)])
        plsc.subcore_barrier()

        def body(it, carry):
            row_off = (wid * n_iter + it) * kch
            pltpu.sync_copy(src_hbm.at[pl.ds(row_off, kch)], srcv)
            pltpu.sync_copy(dst_hbm.at[pl.ds(row_off, kch)], dstv)
            descs = [pltpu.async_copy(x_hbm.at[srcv.at[j]], rows.at[j], gsem)
                     for j in range(kch)]
            for d in descs:
                d.wait()
            descs = [pltpu.async_copy(rows.at[j], aggsh.at[dstv.at[j]], ssem,
                                      add=True)
                     for j in range(kch)]
            for d in descs:
                d.wait()
            return carry

        lax.fori_loop(0, n_iter, body, 0)
        plsc.subcore_barrier()
        pltpu.sync_copy(aggsh.at[pl.ds(si * rpt, rpt)],
                        out_hbm.at[pl.ds(ci * r + si * rpt, rpt)])

    return seg(x_tab, src2, dst2, zeros_tab)


def _atom_lin_body(aggA, aggB, xin, wrel, brel, wroot, a, out_ref):
    agg = aggA[...] + aggB[...]
    y = jnp.dot(agg, wrel[...], preferred_element_type=F32) + brel[...]
    y = y + jnp.dot(xin[...], wroot[...], preferred_element_type=F32)
    out_ref[...] = jnp.where(y >= 0, y, a[...] * y)


def _atom_lin(aggA, aggB, xin, wrel, brel, wroot, a):
    r = xin.shape[0]
    rt = 2048
    full = lambda arr: pl.BlockSpec(arr.shape, lambda i: tuple(0 for _ in arr.shape))
    args = (wrel, brel, wroot, a)
    return pl.pallas_call(
        _atom_lin_body,
        grid=(r // rt,),
        out_shape=jax.ShapeDtypeStruct((r, 64), F32),
        in_specs=[pl.BlockSpec((rt, 64), lambda i: (i, 0))] * 3
                 + [full(arr) for arr in args],
        out_specs=pl.BlockSpec((rt, 64), lambda i: (i, 0)),
    )(aggA, aggB, xin, *args)


def _proj_body(x1, x2, x3, wp, bp_, nref, out_ref):
    rt = x1.shape[0]
    s = x1[...] + x2[...] + x3[...]
    y = jnp.dot(s, wp[...], preferred_element_type=F32) + bp_[...]
    glob = pl.program_id(0) * rt + lax.broadcasted_iota(jnp.int32, (rt, 64), 0)
    out_ref[...] = jnp.where(glob < nref[...], y, 0.0)


def _proj_atom(x1, x2, x3, wp, bp_, n_atom):
    r = x1.shape[0]
    rt = 2048
    nref = jnp.full((1, 1), n_atom, jnp.int32)
    full = lambda arr: pl.BlockSpec(arr.shape, lambda i: tuple(0 for _ in arr.shape))
    return pl.pallas_call(
        _proj_body,
        grid=(r // rt,),
        out_shape=jax.ShapeDtypeStruct((r, 64), F32),
        in_specs=[pl.BlockSpec((rt, 64), lambda i: (i, 0))] * 3
                 + [full(wp), full(bp_), full(nref)],
        out_specs=pl.BlockSpec((rt, 64), lambda i: (i, 0)),
    )(x1, x2, x3, wp, bp_, nref)


def _route_body(gx_ref, x3f_ref, u_ref, out_ref):
    gx = gx_ref[...]
    x3f = x3f_ref[...]
    gn = gx / (jnp.sqrt(jnp.sum(gx * gx, axis=1, keepdims=True)) + 1e-12)
    xn = x3f / (jnp.sqrt(jnp.sum(x3f * x3f, axis=1, keepdims=True)) + 1e-12)
    cos = lax.dot_general(gn, xn, (((1,), (1,)), ((), ())),
                          preferred_element_type=F32)
    m = jnp.max(cos, axis=0, keepdims=True)
    ex = jnp.exp(cos - m)
    den = jnp.sum(ex, axis=0, keepdims=True)
    sm = ex / den
    sm = jnp.where(u_ref[...] < sm, sm, 0.0)
    contrib = jnp.dot(sm, x3f, preferred_element_type=F32)

    @pl.when(pl.program_id(0) == 0)
    def _():
        out_ref[...] = contrib

    @pl.when(pl.program_id(0) != 0)
    def _():
        out_ref[...] = out_ref[...] + contrib


def _route(gx_flat, x3f, u_pad):
    n = gx_flat.shape[0]
    r = x3f.shape[0]
    ct = 1024
    return pl.pallas_call(
        _route_body,
        grid=(r // ct,),
        out_shape=jax.ShapeDtypeStruct((n, 64), F32),
        in_specs=[pl.BlockSpec((n, 64), lambda i: (0, 0)),
                  pl.BlockSpec((ct, 64), lambda i: (i, 0)),
                  pl.BlockSpec((n, ct), lambda i: (0, i))],
        out_specs=pl.BlockSpec((n, 64), lambda i: (0, 0)),
    )(gx_flat, x3f, u_pad)


def _enc_body(gx_ref, nx_ref, w1, w2, b0, out_ref):
    y = jnp.dot(gx_ref[...], w1[...], preferred_element_type=F32)
    y = y + jnp.dot(nx_ref[...], w2[...], preferred_element_type=F32) + b0[...]
    out_ref[...] = y


def _enc(gx_flat, new_x3, p):
    n = gx_flat.shape[0]
    w = p['res_atom_encoder']['W']
    args = (w[:C], w[C:], p['res_atom_encoder']['b'].reshape(1, -1))
    full = lambda arr: pl.BlockSpec(arr.shape, lambda: tuple(0 for _ in arr.shape))
    return pl.pallas_call(
        _enc_body,
        out_shape=jax.ShapeDtypeStruct((n, C), F32),
        in_specs=[full(gx_flat), full(new_x3)] + [full(a) for a in args],
        out_specs=pl.BlockSpec((n, C), lambda: (0, 0)),
    )(gx_flat, new_x3, *args)


def _heads_body(gx_ref, wx, bx, sg, sb, wst, bst, xyz_ref, st_ref):
    gx = gx_ref[...]
    xyz_ref[...] = jnp.dot(gx, wx[...], preferred_element_type=F32) + bx[...]
    gn = _ln(gx, sg[...], sb[...])
    st_ref[...] = jnp.dot(gn, wst[...], preferred_element_type=F32) + bst[...]


def _heads(gx_flat, p):
    n = gx_flat.shape[0]
    args = (p['get_xyz']['W'], p['get_xyz']['b'].reshape(1, -1),
            p['norm_state']['g'].reshape(1, -1), p['norm_state']['be'].reshape(1, -1),
            p['get_state']['W'], p['get_state']['b'].reshape(1, -1))
    d_xyz = p['get_xyz']['W'].shape[1]
    d_st = p['get_state']['W'].shape[1]
    full = lambda arr: pl.BlockSpec(arr.shape, lambda: tuple(0 for _ in arr.shape))
    return pl.pallas_call(
        _heads_body,
        out_shape=(jax.ShapeDtypeStruct((n, d_xyz), F32),
                   jax.ShapeDtypeStruct((n, d_st), F32)),
        in_specs=[full(gx_flat)] + [full(a) for a in args],
        out_specs=(pl.BlockSpec((n, d_xyz), lambda: (0, 0)),
                   pl.BlockSpec((n, d_st), lambda: (0, 0))),
    )(gx_flat, *args)


def kernel(seq1hot, idx, node, edge, x, edge_index, params):
    p = params
    bc, lc = node.shape[0], node.shape[1]
    n = bc * lc

    gx0 = _node_embed(node.reshape(n, -1), seq1hot.reshape(n, -1), p)
    edge_e = _edge_embed(edge, p)
    gx = gx0.reshape(bc, lc, C)
    for bp in p['blocks']:
        gx = _unimp_block(gx, edge_e, bp)

    n_atom = x.shape[0]
    rt = 2048
    r = ((n_atom + rt - 1) // rt) * rt
    x64 = jnp.zeros((r, 64), F32).at[:n_atom, :x.shape[1]].set(x)
    src, dst = edge_index[0], edge_index[1]
    e_num = src.shape[0]
    kch = 8
    n_iter = -(-e_num // (32 * kch * 128))
    e_pad = 32 * n_iter * kch * 128
    src_p = jnp.concatenate([src, jnp.zeros((e_pad - e_num,), jnp.int32)])
    dst_p = jnp.concatenate([dst, jnp.full((e_pad - e_num,), r - 1, jnp.int32)])
    src2 = src_p.reshape(-1, 128)
    dst2 = dst_p.reshape(-1, 128)
    zeros_tab = jnp.zeros((r, 64), F32)

    def pad_w(w):
        return jnp.zeros((64, 64), F32).at[:w.shape[0]].set(w)

    a = p['prelu_a'].reshape(1, 1)
    agg = _sc_segsum(x64, src2, dst2, zeros_tab, n_iter, kch)
    x1 = _atom_lin(agg[:r], agg[r:], x64, pad_w(p['conv1_rel']['W']),
                   p['conv1_rel']['b'].reshape(1, -1), pad_w(p['conv1_root']['W']), a)
    agg = _sc_segsum(x1, src2, dst2, zeros_tab, n_iter, kch)
    x2 = _atom_lin(agg[:r], agg[r:], x1, p['conv2_rel']['W'],
                   p['conv2_rel']['b'].reshape(1, -1), p['conv2_root']['W'], a)
    agg = _sc_segsum(x2, src2, dst2, zeros_tab, n_iter, kch)
    x3 = _atom_lin(agg[:r], agg[r:], x2, p['conv3_rel']['W'],
                   p['conv3_rel']['b'].reshape(1, -1), p['conv3_root']['W'], a)
    x3f = _proj_atom(x1, x2, x3, p['proj_atom']['W'],
                     p['proj_atom']['b'].reshape(1, -1), n_atom)

    u = jax.random.uniform(jax.random.key(42), (n, n_atom), F32)
    u_pad = jnp.ones((n, r), F32).at[:, :n_atom].set(u)
    gx_flat = gx.reshape(n, C)
    new_x3 = _route(gx_flat, x3f, u_pad)

    gx2 = _enc(gx_flat, new_x3, p)
    gx3 = _unimp_block(gx2.reshape(bc, lc, C), edge_e, p['final_block'])
    xyz, state = _heads(gx3.reshape(n, C), p)
    return xyz.reshape(bc, lc, 3, 3), state.reshape(bc, lc, -1)


# speedup vs baseline: 33.7188x; 1.2247x over previous
import functools

import jax
import jax.numpy as jnp
import numpy as np
from jax import lax
from jax.experimental import pallas as pl
from jax.experimental.pallas import tpu as pltpu
from jax.experimental.pallas import tpu_sc as plsc

F32 = jnp.float32
H = 4
C = 64
HC = H * C
EPS = 1e-5


def _ln(x, g, be):
    mu = jnp.mean(x, -1, keepdims=True)
    var = jnp.mean((x - mu) * (x - mu), -1, keepdims=True)
    return g * (x - mu) / jnp.sqrt(var + EPS) + be


def _ln2d(x, g, be):
    d = x.shape[-1]
    ones = jnp.ones((d, 1), F32)
    mu = jnp.dot(x, ones, preferred_element_type=F32) * (1.0 / d)
    xc = x - mu
    var = jnp.dot(xc * xc, ones, preferred_element_type=F32) * (1.0 / d)
    return g * xc / jnp.sqrt(var + EPS) + be


def _node_embed_body(node_ref, seq_ref, ng, nb, wn, ws, b0, g2, b2, out_ref):
    xn = _ln(node_ref[...], ng[...], nb[...])
    y = jnp.dot(xn, wn[...], preferred_element_type=F32)
    y = y + jnp.dot(seq_ref[...], ws[...], preferred_element_type=F32) + b0[...]
    out_ref[...] = _ln(y, g2[...], b2[...])


def _node_embed(node2, seq2, p):
    n = node2.shape[0]
    full = lambda a: pl.BlockSpec(a.shape, lambda: tuple(0 for _ in a.shape))
    args = (node2, seq2,
            p['norm_node']['g'].reshape(1, -1), p['norm_node']['be'].reshape(1, -1),
            p['embed_x_lin']['W'][:node2.shape[1]],
            p['embed_x_lin']['W'][node2.shape[1]:],
            p['embed_x_lin']['b'].reshape(1, -1),
            p['embed_x_ln']['g'].reshape(1, -1), p['embed_x_ln']['be'].reshape(1, -1))
    return pl.pallas_call(
        _node_embed_body,
        out_shape=jax.ShapeDtypeStruct((n, C), F32),
        in_specs=[full(a) for a in args],
        out_specs=pl.BlockSpec((n, C), lambda: (0, 0)),
    )(*args)


def _edge_embed_body(edge_ref, eg, eb, we1, wss, wbn, b0, g2, b2, out_ref):
    it = edge_ref.shape[1]
    l = edge_ref.shape[2]
    e = edge_ref[0]
    en = _ln2d(e.reshape(it * l, -1), eg[...].reshape(1, -1), eb[...].reshape(1, -1))
    proj = jnp.dot(en, we1[...], preferred_element_type=F32)
    proj = proj.reshape(it, l, -1)
    i_glob = pl.program_id(1) * it + lax.broadcasted_iota(jnp.int32, (it, l), 0)
    j_glob = lax.broadcasted_iota(jnp.int32, (it, l), 1)
    s = (j_glob - i_glob).astype(F32)
    sign = jnp.sign(s)
    ss = sign * jnp.clip(jnp.log(jnp.abs(s) + 1.0), 0.0, 5.5)
    bn = jnp.where(jnp.abs(s) > 1.0, 0.0, s)
    y = proj + ss[..., None] * wss[...] + bn[..., None] * wbn[...] + b0[...]
    yn = _ln2d(y.reshape(it * l, -1), g2[...].reshape(1, -1), b2[...].reshape(1, -1))
    out_ref[0] = yn.reshape(it, l, -1)


def _edge_embed(edge, p):
    b, l = edge.shape[0], edge.shape[1]
    ein = edge.shape[3]
    it = 32
    w = p['embed_e_lin']['W']
    args = (p['norm_edge']['g'].reshape(1, 1, -1), p['norm_edge']['be'].reshape(1, 1, -1),
            w[:ein],
            w[ein].reshape(1, 1, -1), w[ein + 1].reshape(1, 1, -1),
            p['embed_e_lin']['b'].reshape(1, 1, -1),
            p['embed_e_ln']['g'].reshape(1, 1, -1), p['embed_e_ln']['be'].reshape(1, 1, -1))
    full = lambda a: pl.BlockSpec(a.shape, lambda bi, ii: tuple(0 for _ in a.shape))
    return pl.pallas_call(
        _edge_embed_body,
        grid=(b, l // it),
        out_shape=jax.ShapeDtypeStruct((b, l, l, C), F32),
        in_specs=[pl.BlockSpec((1, it, l, ein), lambda bi, ii: (bi, ii, 0, 0))]
                 + [full(a) for a in args],
        out_specs=pl.BlockSpec((1, it, l, C), lambda bi, ii: (bi, ii, 0, 0)),
    )(edge, *args)


def _block_body(gxf_ref, gxj_ref, ee_ref, wq, bq, wk, bk, wv, bv, we,
                wskip, bskip, lng, lnb, wlin, blin, out_ref):
    jt = gxj_ref.shape[1]
    l = gxf_ref.shape[1]
    gxf = gxf_ref[0]
    gxj = gxj_ref[0]
    ee = ee_ref[0]
    q = jnp.dot(gxj, wq[...], preferred_element_type=F32) + bq[...]
    k = jnp.dot(gxf, wk[...], preferred_element_type=F32) + bk[...]
    v = jnp.dot(gxf, wv[...], preferred_element_type=F32) + bv[...]
    i_ids = lax.broadcasted_iota(jnp.int32, (l, jt), 0)
    j_ids = pl.program_id(1) * jt + lax.broadcasted_iota(jnp.int32, (l, jt), 1)
    ones_c = jnp.ones((C, 1), F32)
    diag3 = (lax.broadcasted_iota(jnp.int32, (jt, jt, C), 0)
             == lax.broadcasted_iota(jnp.int32, (jt, jt, C), 1))
    ee_flat = ee.reshape(l, jt * C)
    als = []
    for h in range(H):
        sl = slice(h * C, (h + 1) * C)
        qh = q[:, sl]
        kh = k[:, sl]
        weh = we[:, sl]
        qk = lax.dot_general(kh, qh, (((1,), (1,)), ((), ())),
                             preferred_element_type=F32)
        qe = lax.dot_general(qh, weh, (((1,), (1,)), ((), ())),
                             preferred_element_type=F32)
        p_full = ee * qe[None]
        ae_log = jnp.dot(p_full.reshape(l * jt, C), ones_c,
                         preferred_element_type=F32).reshape(l, jt)
        logit = (qk + ae_log) * (1.0 / np.sqrt(C))
        logit = jnp.where(i_ids == j_ids, -1e30, logit)
        m = jnp.max(logit, axis=0, keepdims=True)
        ex = jnp.exp(logit - m)
        den = jnp.sum(ex, axis=0, keepdims=True)
        als.append(ex / (den + 1e-16))
    outs = []
    for h in range(H):
        sl = slice(h * C, (h + 1) * C)
        vh = v[:, sl]
        weh = we[:, sl]
        al = als[h]
        outv = lax.dot_general(al, vh, (((0,), (0,)), ((), ())),
                               preferred_element_type=F32)
        full = lax.dot_general(al, ee_flat, (((0,), (0,)), ((), ())),
                               preferred_element_type=F32)
        ae = jnp.sum(jnp.where(diag3, full.reshape(jt, jt, C), 0.0),
                     axis=0)
        oute = jnp.dot(ae, weh, preferred_element_type=F32)
        outs.append(outv + oute)
    out = jnp.concatenate(outs, axis=1)
    out = out + jnp.dot(gxj, wskip[...], preferred_element_type=F32) + bskip[...]
    out = _ln2d(out, lng[...], lnb[...])
    z = jnp.dot(out, wlin[...], preferred_element_type=F32) + blin[...] + gxj
    out_ref[0] = jnp.where(z > 0, z, jnp.exp(z) - 1.0)


def _unimp_block(gx, edge_e, bp):
    b, l = gx.shape[0], gx.shape[1]
    jt = 64
    args = (bp['q']['W'], bp['q']['b'].reshape(1, -1),
            bp['k']['W'], bp['k']['b'].reshape(1, -1),
            bp['v']['W'], bp['v']['b'].reshape(1, -1),
            bp['e']['W'],
            bp['skip']['W'], bp['skip']['b'].reshape(1, -1),
            bp['ln']['g'].reshape(1, -1), bp['ln']['be'].reshape(1, -1),
            bp['lin']['W'], bp['lin']['b'].reshape(1, -1))
    full = lambda a: pl.BlockSpec(a.shape, lambda bi, ji: tuple(0 for _ in a.shape))
    return pl.pallas_call(
        _block_body,
        grid=(b, l // jt),
        out_shape=jax.ShapeDtypeStruct((b, l, C), F32),
        in_specs=[pl.BlockSpec((1, l, C), lambda bi, ji: (bi, 0, 0)),
                  pl.BlockSpec((1, jt, C), lambda bi, ji: (bi, ji, 0)),
                  pl.BlockSpec((1, l, jt, C), lambda bi, ji: (bi, 0, ji, 0))]
                 + [full(a) for a in args],
        out_specs=pl.BlockSpec((1, jt, C), lambda bi, ji: (bi, ji, 0)),
    )(gx, gx, edge_e, *args)


def _sc_segsum(x_tab, src2, dst2, zeros_tab, n_iter, kch):
    r = x_tab.shape[0]
    rpt = r // 16
    mesh = plsc.VectorSubcoreMesh(core_axis_name="c", subcore_axis_name="s")

    @functools.partial(
        pl.kernel,
        out_type=jax.ShapeDtypeStruct((2 * r, 64), F32),
        mesh=mesh,
        compiler_params=pltpu.CompilerParams(use_tc_tiling_on_sc=False),
        scratch_types=[
            pltpu.VMEM((kch, 128), jnp.int32),
            pltpu.VMEM((kch, 128), jnp.int32),
            pltpu.VMEM((kch, 128, 64), F32),
            pltpu.VMEM_SHARED((r, 64), F32),
            pltpu.SemaphoreType.DMA,
            pltpu.SemaphoreType.DMA,
        ],
    )
    def seg(x_hbm, src_hbm, dst_hbm, zero_hbm, out_hbm,
            srcv, dstv, rows, aggsh, gsem, ssem):
        ci = lax.axis_index("c")
        si = lax.axis_index("s")
        wid = ci * 16 + si
        pltpu.sync_copy(zero_hbm.at[pl.ds(si * rpt, rpt)],
                        aggsh.at[pl.ds(si * rpt, rpt)])
        plsc.subcore_barrier()

        def body(it, carry):
            row_off = (wid * n_iter + it) * kch
            pltpu.sync_copy(src_hbm.at[pl.ds(row_off, kch)], srcv)
            pltpu.sync_copy(dst_hbm.at[pl.ds(row_off, kch)], dstv)
            descs = [pltpu.async_copy(x_hbm.at[srcv.at[j]], rows.at[j], gsem)
                     for j in range(kch)]
            for d in descs:
                d.wait()
            descs = [pltpu.async_copy(rows.at[j], aggsh.at[dstv.at[j]], ssem,
                                      add=True)
                     for j in range(kch)]
            for d in descs:
                d.wait()
            return carry

        lax.fori_loop(0, n_iter, body, 0)
        plsc.subcore_barrier()
        pltpu.sync_copy(aggsh.at[pl.ds(si * rpt, rpt)],
                        out_hbm.at[pl.ds(ci * r + si * rpt, rpt)])

    return seg(x_tab, src2, dst2, zeros_tab)


def _atom_lin_body(aggA, aggB, xin, wrel, brel, wroot, a, out_ref):
    agg = aggA[...] + aggB[...]
    y = jnp.dot(agg, wrel[...], preferred_element_type=F32) + brel[...]
    y = y + jnp.dot(xin[...], wroot[...], preferred_element_type=F32)
    out_ref[...] = jnp.where(y >= 0, y, a[...] * y)


def _atom_lin(aggA, aggB, xin, wrel, brel, wroot, a):
    r = xin.shape[0]
    rt = 2048
    full = lambda arr: pl.BlockSpec(arr.shape, lambda i: tuple(0 for _ in arr.shape))
    args = (wrel, brel, wroot, a)
    return pl.pallas_call(
        _atom_lin_body,
        grid=(r // rt,),
        out_shape=jax.ShapeDtypeStruct((r, 64), F32),
        in_specs=[pl.BlockSpec((rt, 64), lambda i: (i, 0))] * 3
                 + [full(arr) for arr in args],
        out_specs=pl.BlockSpec((rt, 64), lambda i: (i, 0)),
    )(aggA, aggB, xin, *args)


def _proj_body(x1, x2, x3, wp, bp_, nref, out_ref):
    rt = x1.shape[0]
    s = x1[...] + x2[...] + x3[...]
    y = jnp.dot(s, wp[...], preferred_element_type=F32) + bp_[...]
    glob = pl.program_id(0) * rt + lax.broadcasted_iota(jnp.int32, (rt, 64), 0)
    out_ref[...] = jnp.where(glob < nref[...], y, 0.0)


def _proj_atom(x1, x2, x3, wp, bp_, n_atom):
    r = x1.shape[0]
    rt = 2048
    nref = jnp.full((1, 1), n_atom, jnp.int32)
    full = lambda arr: pl.BlockSpec(arr.shape, lambda i: tuple(0 for _ in arr.shape))
    return pl.pallas_call(
        _proj_body,
        grid=(r // rt,),
        out_shape=jax.ShapeDtypeStruct((r, 64), F32),
        in_specs=[pl.BlockSpec((rt, 64), lambda i: (i, 0))] * 3
                 + [full(wp), full(bp_), full(nref)],
        out_specs=pl.BlockSpec((rt, 64), lambda i: (i, 0)),
    )(x1, x2, x3, wp, bp_, nref)


def _route_body(gx_ref, x3f_ref, u_ref, out_ref):
    gx = gx_ref[...]
    x3f = x3f_ref[...]
    gn = gx / (jnp.sqrt(jnp.sum(gx * gx, axis=1, keepdims=True)) + 1e-12)
    xn = x3f / (jnp.sqrt(jnp.sum(x3f * x3f, axis=1, keepdims=True)) + 1e-12)
    cos = lax.dot_general(gn, xn, (((1,), (1,)), ((), ())),
                          preferred_element_type=F32)
    m = jnp.max(cos, axis=0, keepdims=True)
    ex = jnp.exp(cos - m)
    den = jnp.sum(ex, axis=0, keepdims=True)
    sm = ex / den
    sm = jnp.where(u_ref[...] < sm, sm, 0.0)
    contrib = jnp.dot(sm, x3f, preferred_element_type=F32)

    @pl.when(pl.program_id(0) == 0)
    def _():
        out_ref[...] = contrib

    @pl.when(pl.program_id(0) != 0)
    def _():
        out_ref[...] = out_ref[...] + contrib


def _route(gx_flat, x3f, u_pad):
    n = gx_flat.shape[0]
    r = x3f.shape[0]
    ct = 1024
    return pl.pallas_call(
        _route_body,
        grid=(r // ct,),
        out_shape=jax.ShapeDtypeStruct((n, 64), F32),
        in_specs=[pl.BlockSpec((n, 64), lambda i: (0, 0)),
                  pl.BlockSpec((ct, 64), lambda i: (i, 0)),
                  pl.BlockSpec((n, ct), lambda i: (0, i))],
        out_specs=pl.BlockSpec((n, 64), lambda i: (0, 0)),
    )(gx_flat, x3f, u_pad)


def _enc_body(gx_ref, nx_ref, w1, w2, b0, out_ref):
    y = jnp.dot(gx_ref[...], w1[...], preferred_element_type=F32)
    y = y + jnp.dot(nx_ref[...], w2[...], preferred_element_type=F32) + b0[...]
    out_ref[...] = y


def _enc(gx_flat, new_x3, p):
    n = gx_flat.shape[0]
    w = p['res_atom_encoder']['W']
    args = (w[:C], w[C:], p['res_atom_encoder']['b'].reshape(1, -1))
    full = lambda arr: pl.BlockSpec(arr.shape, lambda: tuple(0 for _ in arr.shape))
    return pl.pallas_call(
        _enc_body,
        out_shape=jax.ShapeDtypeStruct((n, C), F32),
        in_specs=[full(gx_flat), full(new_x3)] + [full(a) for a in args],
        out_specs=pl.BlockSpec((n, C), lambda: (0, 0)),
    )(gx_flat, new_x3, *args)


def _heads_body(gx_ref, wx, bx, sg, sb, wst, bst, xyz_ref, st_ref):
    gx = gx_ref[...]
    xyz_ref[...] = jnp.dot(gx, wx[...], preferred_element_type=F32) + bx[...]
    gn = _ln(gx, sg[...], sb[...])
    st_ref[...] = jnp.dot(gn, wst[...], preferred_element_type=F32) + bst[...]


def _heads(gx_flat, p):
    n = gx_flat.shape[0]
    args = (p['get_xyz']['W'], p['get_xyz']['b'].reshape(1, -1),
            p['norm_state']['g'].reshape(1, -1), p['norm_state']['be'].reshape(1, -1),
            p['get_state']['W'], p['get_state']['b'].reshape(1, -1))
    d_xyz = p['get_xyz']['W'].shape[1]
    d_st = p['get_state']['W'].shape[1]
    full = lambda arr: pl.BlockSpec(arr.shape, lambda: tuple(0 for _ in arr.shape))
    return pl.pallas_call(
        _heads_body,
        out_shape=(jax.ShapeDtypeStruct((n, d_xyz), F32),
                   jax.ShapeDtypeStruct((n, d_st), F32)),
        in_specs=[full(gx_flat)] + [full(a) for a in args],
        out_specs=(pl.BlockSpec((n, d_xyz), lambda: (0, 0)),
                   pl.BlockSpec((n, d_st), lambda: (0, 0))),
    )(gx_flat, *args)


def kernel(seq1hot, idx, node, edge, x, edge_index, params):
    p = params
    bc, lc = node.shape[0], node.shape[1]
    n = bc * lc

    gx0 = _node_embed(node.reshape(n, -1), seq1hot.reshape(n, -1), p)
    edge_e = _edge_embed(edge, p)
    gx = gx0.reshape(bc, lc, C)
    for bp in p['blocks']:
        gx = _unimp_block(gx, edge_e, bp)

    n_atom = x.shape[0]
    rt = 2048
    r = ((n_atom + rt - 1) // rt) * rt
    x64 = jnp.zeros((r, 64), F32).at[:n_atom, :x.shape[1]].set(x)
    src, dst = edge_index[0], edge_index[1]
    e_num = src.shape[0]
    kch = 8
    n_iter = -(-e_num // (32 * kch * 128))
    e_pad = 32 * n_iter * kch * 128
    src_p = jnp.concatenate([src, jnp.zeros((e_pad - e_num,), jnp.int32)])
    dst_p = jnp.concatenate([dst, jnp.full((e_pad - e_num,), r - 1, jnp.int32)])
    src2 = src_p.reshape(-1, 128)
    dst2 = dst_p.reshape(-1, 128)
    zeros_tab = jnp.zeros((r, 64), F32)

    def pad_w(w):
        return jnp.zeros((64, 64), F32).at[:w.shape[0]].set(w)

    a = p['prelu_a'].reshape(1, 1)
    agg = _sc_segsum(x64, src2, dst2, zeros_tab, n_iter, kch)
    x1 = _atom_lin(agg[:r], agg[r:], x64, pad_w(p['conv1_rel']['W']),
                   p['conv1_rel']['b'].reshape(1, -1), pad_w(p['conv1_root']['W']), a)
    agg = _sc_segsum(x1, src2, dst2, zeros_tab, n_iter, kch)
    x2 = _atom_lin(agg[:r], agg[r:], x1, p['conv2_rel']['W'],
                   p['conv2_rel']['b'].reshape(1, -1), p['conv2_root']['W'], a)
    agg = _sc_segsum(x2, src2, dst2, zeros_tab, n_iter, kch)
    x3 = _atom_lin(agg[:r], agg[r:], x2, p['conv3_rel']['W'],
                   p['conv3_rel']['b'].reshape(1, -1), p['conv3_root']['W'], a)
    x3f = _proj_atom(x1, x2, x3, p['proj_atom']['W'],
                     p['proj_atom']['b'].reshape(1, -1), n_atom)

    u = jax.random.uniform(jax.random.key(42), (n, n_atom), F32)
    u_pad = jnp.ones((n, r), F32).at[:, :n_atom].set(u)
    gx_flat = gx.reshape(n, C)
    new_x3 = _route(gx_flat, x3f, u_pad)

    gx2 = _enc(gx_flat, new_x3, p)
    gx3 = _unimp_block(gx2.reshape(bc, lc, C), edge_e, p['final_block'])
    xyz, state = _heads(gx3.reshape(n, C), p)
    return xyz.reshape(bc, lc, 3, 3), state.reshape(bc, lc, -1)


# speedup vs baseline: 34.5738x; 1.0254x over previous
import functools

import jax
import jax.numpy as jnp
import numpy as np
from jax import lax
from jax.experimental import pallas as pl
from jax.experimental.pallas import tpu as pltpu
from jax.experimental.pallas import tpu_sc as plsc

F32 = jnp.float32
H = 4
C = 64
HC = H * C
EPS = 1e-5


def _ln(x, g, be):
    mu = jnp.mean(x, -1, keepdims=True)
    var = jnp.mean((x - mu) * (x - mu), -1, keepdims=True)
    return g * (x - mu) / jnp.sqrt(var + EPS) + be


def _ln2d(x, g, be):
    d = x.shape[-1]
    ones = jnp.ones((d, 1), F32)
    mu = jnp.dot(x, ones, preferred_element_type=F32) * (1.0 / d)
    xc = x - mu
    var = jnp.dot(xc * xc, ones, preferred_element_type=F32) * (1.0 / d)
    return g * xc / jnp.sqrt(var + EPS) + be


def _node_embed_body(node_ref, seq_ref, ng, nb, wn, ws, b0, g2, b2, out_ref):
    xn = _ln(node_ref[...], ng[...], nb[...])
    y = jnp.dot(xn, wn[...], preferred_element_type=F32)
    y = y + jnp.dot(seq_ref[...], ws[...], preferred_element_type=F32) + b0[...]
    out_ref[...] = _ln(y, g2[...], b2[...])


def _node_embed(node2, seq2, p):
    n = node2.shape[0]
    full = lambda a: pl.BlockSpec(a.shape, lambda: tuple(0 for _ in a.shape))
    args = (node2, seq2,
            p['norm_node']['g'].reshape(1, -1), p['norm_node']['be'].reshape(1, -1),
            p['embed_x_lin']['W'][:node2.shape[1]],
            p['embed_x_lin']['W'][node2.shape[1]:],
            p['embed_x_lin']['b'].reshape(1, -1),
            p['embed_x_ln']['g'].reshape(1, -1), p['embed_x_ln']['be'].reshape(1, -1))
    return pl.pallas_call(
        _node_embed_body,
        out_shape=jax.ShapeDtypeStruct((n, C), F32),
        in_specs=[full(a) for a in args],
        out_specs=pl.BlockSpec((n, C), lambda: (0, 0)),
    )(*args)


def _edge_embed_body(edge_ref, eg, eb, we1, wss, wbn, b0, g2, b2, out_ref):
    it = edge_ref.shape[1]
    l = edge_ref.shape[2]
    e = edge_ref[0]
    en = _ln2d(e.reshape(it * l, -1), eg[...].reshape(1, -1), eb[...].reshape(1, -1))
    proj = jnp.dot(en, we1[...], preferred_element_type=F32)
    proj = proj.reshape(it, l, -1)
    i_glob = pl.program_id(1) * it + lax.broadcasted_iota(jnp.int32, (it, l), 0)
    j_glob = lax.broadcasted_iota(jnp.int32, (it, l), 1)
    s = (j_glob - i_glob).astype(F32)
    sign = jnp.sign(s)
    ss = sign * jnp.clip(jnp.log(jnp.abs(s) + 1.0), 0.0, 5.5)
    bn = jnp.where(jnp.abs(s) > 1.0, 0.0, s)
    y = proj + ss[..., None] * wss[...] + bn[..., None] * wbn[...] + b0[...]
    yn = _ln2d(y.reshape(it * l, -1), g2[...].reshape(1, -1), b2[...].reshape(1, -1))
    out_ref[0] = yn.reshape(it, l, -1)


def _edge_embed(edge, p):
    b, l = edge.shape[0], edge.shape[1]
    ein = edge.shape[3]
    it = 32
    w = p['embed_e_lin']['W']
    args = (p['norm_edge']['g'].reshape(1, 1, -1), p['norm_edge']['be'].reshape(1, 1, -1),
            w[:ein],
            w[ein].reshape(1, 1, -1), w[ein + 1].reshape(1, 1, -1),
            p['embed_e_lin']['b'].reshape(1, 1, -1),
            p['embed_e_ln']['g'].reshape(1, 1, -1), p['embed_e_ln']['be'].reshape(1, 1, -1))
    full = lambda a: pl.BlockSpec(a.shape, lambda bi, ii: tuple(0 for _ in a.shape))
    return pl.pallas_call(
        _edge_embed_body,
        grid=(b, l // it),
        out_shape=jax.ShapeDtypeStruct((b, l, l, C), F32),
        in_specs=[pl.BlockSpec((1, it, l, ein), lambda bi, ii: (bi, ii, 0, 0))]
                 + [full(a) for a in args],
        out_specs=pl.BlockSpec((1, it, l, C), lambda bi, ii: (bi, ii, 0, 0)),
    )(edge, *args)


def _block_body(gxf_ref, gxj_ref, ee_ref, wq, bq, wk, bk, wv, bv, we,
                wskip, bskip, lng, lnb, wlin, blin, out_ref):
    jt = gxj_ref.shape[1]
    l = gxf_ref.shape[1]
    gxf = gxf_ref[0]
    gxj = gxj_ref[0]
    ee = ee_ref[0]
    q = jnp.dot(gxj, wq[...], preferred_element_type=F32) + bq[...]
    k = jnp.dot(gxf, wk[...], preferred_element_type=F32) + bk[...]
    v = jnp.dot(gxf, wv[...], preferred_element_type=F32) + bv[...]
    i_ids = lax.broadcasted_iota(jnp.int32, (l, jt), 0)
    j_ids = pl.program_id(1) * jt + lax.broadcasted_iota(jnp.int32, (l, jt), 1)
    ones_c = jnp.ones((C, 1), F32)
    diag3 = (lax.broadcasted_iota(jnp.int32, (jt, jt, C), 0)
             == lax.broadcasted_iota(jnp.int32, (jt, jt, C), 1))
    ee_flat = ee.reshape(l, jt * C)
    als = []
    for h in range(H):
        sl = slice(h * C, (h + 1) * C)
        qh = q[:, sl]
        kh = k[:, sl]
        weh = we[:, sl]
        qk = lax.dot_general(kh, qh, (((1,), (1,)), ((), ())),
                             preferred_element_type=F32)
        qe = lax.dot_general(qh, weh, (((1,), (1,)), ((), ())),
                             preferred_element_type=F32)
        p_full = ee * qe[None]
        ae_log = jnp.dot(p_full.reshape(l * jt, C), ones_c,
                         preferred_element_type=F32).reshape(l, jt)
        logit = (qk + ae_log) * (1.0 / np.sqrt(C))
        logit = jnp.where(i_ids == j_ids, -1e30, logit)
        m = jnp.max(logit, axis=0, keepdims=True)
        ex = jnp.exp(logit - m)
        den = jnp.sum(ex, axis=0, keepdims=True)
        als.append(ex / (den + 1e-16))
    outs = []
    for h in range(H):
        sl = slice(h * C, (h + 1) * C)
        vh = v[:, sl]
        weh = we[:, sl]
        al = als[h]
        outv = lax.dot_general(al, vh, (((0,), (0,)), ((), ())),
                               preferred_element_type=F32)
        full = lax.dot_general(al, ee_flat, (((0,), (0,)), ((), ())),
                               preferred_element_type=F32)
        ae = jnp.sum(jnp.where(diag3, full.reshape(jt, jt, C), 0.0),
                     axis=0)
        oute = jnp.dot(ae, weh, preferred_element_type=F32)
        outs.append(outv + oute)
    out = jnp.concatenate(outs, axis=1)
    out = out + jnp.dot(gxj, wskip[...], preferred_element_type=F32) + bskip[...]
    out = _ln2d(out, lng[...], lnb[...])
    z = jnp.dot(out, wlin[...], preferred_element_type=F32) + blin[...] + gxj
    out_ref[0] = jnp.where(z > 0, z, jnp.exp(z) - 1.0)


def _unimp_block(gx, edge_e, bp):
    b, l = gx.shape[0], gx.shape[1]
    jt = 64
    args = (bp['q']['W'], bp['q']['b'].reshape(1, -1),
            bp['k']['W'], bp['k']['b'].reshape(1, -1),
            bp['v']['W'], bp['v']['b'].reshape(1, -1),
            bp['e']['W'],
            bp['skip']['W'], bp['skip']['b'].reshape(1, -1),
            bp['ln']['g'].reshape(1, -1), bp['ln']['be'].reshape(1, -1),
            bp['lin']['W'], bp['lin']['b'].reshape(1, -1))
    full = lambda a: pl.BlockSpec(a.shape, lambda bi, ji: tuple(0 for _ in a.shape))
    return pl.pallas_call(
        _block_body,
        grid=(b, l // jt),
        out_shape=jax.ShapeDtypeStruct((b, l, C), F32),
        in_specs=[pl.BlockSpec((1, l, C), lambda bi, ji: (bi, 0, 0)),
                  pl.BlockSpec((1, jt, C), lambda bi, ji: (bi, ji, 0)),
                  pl.BlockSpec((1, l, jt, C), lambda bi, ji: (bi, 0, ji, 0))]
                 + [full(a) for a in args],
        out_specs=pl.BlockSpec((1, jt, C), lambda bi, ji: (bi, ji, 0)),
    )(gx, gx, edge_e, *args)


def _sc_segsum(x_tab, src2, dst2, zeros_tab, n_iter, kch):
    r = x_tab.shape[0]
    rpt = r // 16
    n_rows = n_iter * kch
    n_grp = n_rows // (2 * kch)
    mesh = plsc.VectorSubcoreMesh(core_axis_name="c", subcore_axis_name="s")

    @functools.partial(
        pl.kernel,
        out_type=jax.ShapeDtypeStruct((2 * r, 64), F32),
        mesh=mesh,
        compiler_params=pltpu.CompilerParams(use_tc_tiling_on_sc=False),
        scratch_types=[
            pltpu.VMEM((n_rows, 128), jnp.int32),
            pltpu.VMEM((n_rows, 128), jnp.int32),
            pltpu.VMEM((2, kch, 128, 64), F32),
            pltpu.VMEM_SHARED((r, 64), F32),
            pltpu.SemaphoreType.DMA,
            pltpu.SemaphoreType.DMA,
        ],
    )
    def seg(x_hbm, src_hbm, dst_hbm, zero_hbm, out_hbm,
            srcv, dstv, rows, aggsh, gsem, ssem):
        ci = lax.axis_index("c")
        si = lax.axis_index("s")
        wid = ci * 16 + si
        pltpu.sync_copy(src_hbm.at[pl.ds(wid * n_rows, n_rows)], srcv)
        pltpu.sync_copy(dst_hbm.at[pl.ds(wid * n_rows, n_rows)], dstv)
        pltpu.sync_copy(zero_hbm.at[pl.ds(si * rpt, rpt)],
                        aggsh.at[pl.ds(si * rpt, rpt)])
        plsc.subcore_barrier()

        def body(gg, carry):
            base = gg * 2 * kch
            gd = []
            for par in (0, 1):
                gd.append([pltpu.async_copy(
                    x_hbm.at[srcv.at[base + par * kch + j]],
                    rows.at[par, j], gsem) for j in range(kch)])
            sd = []
            for par in (0, 1):
                for d in gd[par]:
                    d.wait()
                sd += [pltpu.async_copy(
                    rows.at[par, j], aggsh.at[dstv.at[base + par * kch + j]],
                    ssem, add=True) for j in range(kch)]
            for d in sd:
                d.wait()
            return carry

        lax.fori_loop(0, n_grp, body, 0)
        plsc.subcore_barrier()
        pltpu.sync_copy(aggsh.at[pl.ds(si * rpt, rpt)],
                        out_hbm.at[pl.ds(ci * r + si * rpt, rpt)])

    return seg(x_tab, src2, dst2, zeros_tab)


def _atom_lin_body(aggA, aggB, xin, wrel, brel, wroot, a, out_ref):
    agg = aggA[...] + aggB[...]
    y = jnp.dot(agg, wrel[...], preferred_element_type=F32) + brel[...]
    y = y + jnp.dot(xin[...], wroot[...], preferred_element_type=F32)
    out_ref[...] = jnp.where(y >= 0, y, a[...] * y)


def _atom_lin(aggA, aggB, xin, wrel, brel, wroot, a):
    r = xin.shape[0]
    rt = 2048
    full = lambda arr: pl.BlockSpec(arr.shape, lambda i: tuple(0 for _ in arr.shape))
    args = (wrel, brel, wroot, a)
    return pl.pallas_call(
        _atom_lin_body,
        grid=(r // rt,),
        out_shape=jax.ShapeDtypeStruct((r, 64), F32),
        in_specs=[pl.BlockSpec((rt, 64), lambda i: (i, 0))] * 3
                 + [full(arr) for arr in args],
        out_specs=pl.BlockSpec((rt, 64), lambda i: (i, 0)),
    )(aggA, aggB, xin, *args)


def _proj_body(x1, x2, x3, wp, bp_, nref, out_ref):
    rt = x1.shape[0]
    s = x1[...] + x2[...] + x3[...]
    y = jnp.dot(s, wp[...], preferred_element_type=F32) + bp_[...]
    glob = pl.program_id(0) * rt + lax.broadcasted_iota(jnp.int32, (rt, 64), 0)
    out_ref[...] = jnp.where(glob < nref[...], y, 0.0)


def _proj_atom(x1, x2, x3, wp, bp_, n_atom):
    r = x1.shape[0]
    rt = 2048
    nref = jnp.full((1, 1), n_atom, jnp.int32)
    full = lambda arr: pl.BlockSpec(arr.shape, lambda i: tuple(0 for _ in arr.shape))
    return pl.pallas_call(
        _proj_body,
        grid=(r // rt,),
        out_shape=jax.ShapeDtypeStruct((r, 64), F32),
        in_specs=[pl.BlockSpec((rt, 64), lambda i: (i, 0))] * 3
                 + [full(wp), full(bp_), full(nref)],
        out_specs=pl.BlockSpec((rt, 64), lambda i: (i, 0)),
    )(x1, x2, x3, wp, bp_, nref)


def _route_body(gx_ref, x3f_ref, u_ref, out_ref):
    gx = gx_ref[...]
    x3f = x3f_ref[...]
    gn = gx / (jnp.sqrt(jnp.sum(gx * gx, axis=1, keepdims=True)) + 1e-12)
    xn = x3f / (jnp.sqrt(jnp.sum(x3f * x3f, axis=1, keepdims=True)) + 1e-12)
    cos = lax.dot_general(gn, xn, (((1,), (1,)), ((), ())),
                          preferred_element_type=F32)
    m = jnp.max(cos, axis=0, keepdims=True)
    ex = jnp.exp(cos - m)
    den = jnp.sum(ex, axis=0, keepdims=True)
    sm = ex / den
    sm = jnp.where(u_ref[...] < sm, sm, 0.0)
    contrib = jnp.dot(sm, x3f, preferred_element_type=F32)

    @pl.when(pl.program_id(0) == 0)
    def _():
        out_ref[...] = contrib

    @pl.when(pl.program_id(0) != 0)
    def _():
        out_ref[...] = out_ref[...] + contrib


def _route(gx_flat, x3f, u_pad):
    n = gx_flat.shape[0]
    r = x3f.shape[0]
    ct = 1024
    return pl.pallas_call(
        _route_body,
        grid=(r // ct,),
        out_shape=jax.ShapeDtypeStruct((n, 64), F32),
        in_specs=[pl.BlockSpec((n, 64), lambda i: (0, 0)),
                  pl.BlockSpec((ct, 64), lambda i: (i, 0)),
                  pl.BlockSpec((n, ct), lambda i: (0, i))],
        out_specs=pl.BlockSpec((n, 64), lambda i: (0, 0)),
    )(gx_flat, x3f, u_pad)


def _enc_body(gx_ref, nx_ref, w1, w2, b0, out_ref):
    y = jnp.dot(gx_ref[...], w1[...], preferred_element_type=F32)
    y = y + jnp.dot(nx_ref[...], w2[...], preferred_element_type=F32) + b0[...]
    out_ref[...] = y


def _enc(gx_flat, new_x3, p):
    n = gx_flat.shape[0]
    w = p['res_atom_encoder']['W']
    args = (w[:C], w[C:], p['res_atom_encoder']['b'].reshape(1, -1))
    full = lambda arr: pl.BlockSpec(arr.shape, lambda: tuple(0 for _ in arr.shape))
    return pl.pallas_call(
        _enc_body,
        out_shape=jax.ShapeDtypeStruct((n, C), F32),
        in_specs=[full(gx_flat), full(new_x3)] + [full(a) for a in args],
        out_specs=pl.BlockSpec((n, C), lambda: (0, 0)),
    )(gx_flat, new_x3, *args)


def _heads_body(gx_ref, wx, bx, sg, sb, wst, bst, xyz_ref, st_ref):
    gx = gx_ref[...]
    xyz_ref[...] = jnp.dot(gx, wx[...], preferred_element_type=F32) + bx[...]
    gn = _ln(gx, sg[...], sb[...])
    st_ref[...] = jnp.dot(gn, wst[...], preferred_element_type=F32) + bst[...]


def _heads(gx_flat, p):
    n = gx_flat.shape[0]
    args = (p['get_xyz']['W'], p['get_xyz']['b'].reshape(1, -1),
            p['norm_state']['g'].reshape(1, -1), p['norm_state']['be'].reshape(1, -1),
            p['get_state']['W'], p['get_state']['b'].reshape(1, -1))
    d_xyz = p['get_xyz']['W'].shape[1]
    d_st = p['get_state']['W'].shape[1]
    full = lambda arr: pl.BlockSpec(arr.shape, lambda: tuple(0 for _ in arr.shape))
    return pl.pallas_call(
        _heads_body,
        out_shape=(jax.ShapeDtypeStruct((n, d_xyz), F32),
                   jax.ShapeDtypeStruct((n, d_st), F32)),
        in_specs=[full(gx_flat)] + [full(a) for a in args],
        out_specs=(pl.BlockSpec((n, d_xyz), lambda: (0, 0)),
                   pl.BlockSpec((n, d_st), lambda: (0, 0))),
    )(gx_flat, *args)


def kernel(seq1hot, idx, node, edge, x, edge_index, params):
    p = params
    bc, lc = node.shape[0], node.shape[1]
    n = bc * lc

    gx0 = _node_embed(node.reshape(n, -1), seq1hot.reshape(n, -1), p)
    edge_e = _edge_embed(edge, p)
    gx = gx0.reshape(bc, lc, C)
    for bp in p['blocks']:
        gx = _unimp_block(gx, edge_e, bp)

    n_atom = x.shape[0]
    rt = 2048
    r = ((n_atom + rt - 1) // rt) * rt
    x64 = jnp.zeros((r, 64), F32).at[:n_atom, :x.shape[1]].set(x)
    src, dst = edge_index[0], edge_index[1]
    e_num = src.shape[0]
    kch = 4
    n_rows_t = -(-e_num // (32 * 128))
    n_rows_t = -(-n_rows_t // (2 * kch)) * (2 * kch)
    n_iter = n_rows_t // kch
    e_pad = 32 * n_rows_t * 128
    src_p = jnp.concatenate([src, jnp.zeros((e_pad - e_num,), jnp.int32)])
    dst_p = jnp.concatenate([dst, jnp.full((e_pad - e_num,), r - 1, jnp.int32)])
    src2 = src_p.reshape(-1, 128)
    dst2 = dst_p.reshape(-1, 128)
    zeros_tab = jnp.zeros((r, 64), F32)

    def pad_w(w):
        return jnp.zeros((64, 64), F32).at[:w.shape[0]].set(w)

    a = p['prelu_a'].reshape(1, 1)
    agg = _sc_segsum(x64, src2, dst2, zeros_tab, n_iter, kch)
    x1 = _atom_lin(agg[:r], agg[r:], x64, pad_w(p['conv1_rel']['W']),
                   p['conv1_rel']['b'].reshape(1, -1), pad_w(p['conv1_root']['W']), a)
    agg = _sc_segsum(x1, src2, dst2, zeros_tab, n_iter, kch)
    x2 = _atom_lin(agg[:r], agg[r:], x1, p['conv2_rel']['W'],
                   p['conv2_rel']['b'].reshape(1, -1), p['conv2_root']['W'], a)
    agg = _sc_segsum(x2, src2, dst2, zeros_tab, n_iter, kch)
    x3 = _atom_lin(agg[:r], agg[r:], x2, p['conv3_rel']['W'],
                   p['conv3_rel']['b'].reshape(1, -1), p['conv3_root']['W'], a)
    x3f = _proj_atom(x1, x2, x3, p['proj_atom']['W'],
                     p['proj_atom']['b'].reshape(1, -1), n_atom)

    u = jax.random.uniform(jax.random.key(42), (n, n_atom), F32)
    u_pad = jnp.ones((n, r), F32).at[:, :n_atom].set(u)
    gx_flat = gx.reshape(n, C)
    new_x3 = _route(gx_flat, x3f, u_pad)

    gx2 = _enc(gx_flat, new_x3, p)
    gx3 = _unimp_block(gx2.reshape(bc, lc, C), edge_e, p['final_block'])
    xyz, state = _heads(gx3.reshape(n, C), p)
    return xyz.reshape(bc, lc, 3, 3), state.reshape(bc, lc, -1)


# speedup vs baseline: 36.2156x; 1.0475x over previous
import functools

import jax
import jax.numpy as jnp
import numpy as np
from jax import lax
from jax.experimental import pallas as pl
from jax.experimental.pallas import tpu as pltpu
from jax.experimental.pallas import tpu_sc as plsc

F32 = jnp.float32
H = 4
C = 64
HC = H * C
EPS = 1e-5


def _ln(x, g, be):
    mu = jnp.mean(x, -1, keepdims=True)
    var = jnp.mean((x - mu) * (x - mu), -1, keepdims=True)
    return g * (x - mu) / jnp.sqrt(var + EPS) + be


def _ln2d(x, g, be):
    d = x.shape[-1]
    ones = jnp.ones((d, 1), F32)
    mu = jnp.dot(x, ones, preferred_element_type=F32) * (1.0 / d)
    xc = x - mu
    var = jnp.dot(xc * xc, ones, preferred_element_type=F32) * (1.0 / d)
    return g * xc / jnp.sqrt(var + EPS) + be


def _node_embed_body(node_ref, seq_ref, ng, nb, wn, ws, b0, g2, b2, out_ref):
    xn = _ln(node_ref[...], ng[...], nb[...])
    y = jnp.dot(xn, wn[...], preferred_element_type=F32)
    y = y + jnp.dot(seq_ref[...], ws[...], preferred_element_type=F32) + b0[...]
    out_ref[...] = _ln(y, g2[...], b2[...])


def _node_embed(node2, seq2, p):
    n = node2.shape[0]
    full = lambda a: pl.BlockSpec(a.shape, lambda: tuple(0 for _ in a.shape))
    args = (node2, seq2,
            p['norm_node']['g'].reshape(1, -1), p['norm_node']['be'].reshape(1, -1),
            p['embed_x_lin']['W'][:node2.shape[1]],
            p['embed_x_lin']['W'][node2.shape[1]:],
            p['embed_x_lin']['b'].reshape(1, -1),
            p['embed_x_ln']['g'].reshape(1, -1), p['embed_x_ln']['be'].reshape(1, -1))
    return pl.pallas_call(
        _node_embed_body,
        out_shape=jax.ShapeDtypeStruct((n, C), F32),
        in_specs=[full(a) for a in args],
        out_specs=pl.BlockSpec((n, C), lambda: (0, 0)),
    )(*args)


def _edge_embed_body(edge_ref, eg, eb, we1, wss, wbn, b0, g2, b2, out_ref):
    it = edge_ref.shape[1]
    l = edge_ref.shape[2]
    e = edge_ref[0]
    en = _ln2d(e.reshape(it * l, -1), eg[...].reshape(1, -1), eb[...].reshape(1, -1))
    proj = jnp.dot(en, we1[...], preferred_element_type=F32)
    proj = proj.reshape(it, l, -1)
    i_glob = pl.program_id(1) * it + lax.broadcasted_iota(jnp.int32, (it, l), 0)
    j_glob = lax.broadcasted_iota(jnp.int32, (it, l), 1)
    s = (j_glob - i_glob).astype(F32)
    sign = jnp.sign(s)
    ss = sign * jnp.clip(jnp.log(jnp.abs(s) + 1.0), 0.0, 5.5)
    bn = jnp.where(jnp.abs(s) > 1.0, 0.0, s)
    y = proj + ss[..., None] * wss[...] + bn[..., None] * wbn[...] + b0[...]
    yn = _ln2d(y.reshape(it * l, -1), g2[...].reshape(1, -1), b2[...].reshape(1, -1))
    out_ref[0] = yn.reshape(it, l, -1).astype(jnp.bfloat16)


def _edge_embed(edge, p):
    b, l = edge.shape[0], edge.shape[1]
    ein = edge.shape[3]
    it = 32
    w = p['embed_e_lin']['W']
    args = (p['norm_edge']['g'].reshape(1, 1, -1), p['norm_edge']['be'].reshape(1, 1, -1),
            w[:ein],
            w[ein].reshape(1, 1, -1), w[ein + 1].reshape(1, 1, -1),
            p['embed_e_lin']['b'].reshape(1, 1, -1),
            p['embed_e_ln']['g'].reshape(1, 1, -1), p['embed_e_ln']['be'].reshape(1, 1, -1))
    full = lambda a: pl.BlockSpec(a.shape, lambda bi, ii: tuple(0 for _ in a.shape))
    return pl.pallas_call(
        _edge_embed_body,
        grid=(b, l // it),
        out_shape=jax.ShapeDtypeStruct((b, l, l, C), jnp.bfloat16),
        in_specs=[pl.BlockSpec((1, it, l, ein), lambda bi, ii: (bi, ii, 0, 0))]
                 + [full(a) for a in args],
        out_specs=pl.BlockSpec((1, it, l, C), lambda bi, ii: (bi, ii, 0, 0)),
    )(edge, *args)


def _block_body(gxf_ref, gxj_ref, ee_ref, wq, bq, wk, bk, wv, bv, we,
                wskip, bskip, lng, lnb, wlin, blin, out_ref):
    jt = gxj_ref.shape[1]
    l = gxf_ref.shape[1]
    gxf = gxf_ref[0]
    gxj = gxj_ref[0]
    ee = ee_ref[0]
    q = jnp.dot(gxj, wq[...], preferred_element_type=F32) + bq[...]
    k = jnp.dot(gxf, wk[...], preferred_element_type=F32) + bk[...]
    v = jnp.dot(gxf, wv[...], preferred_element_type=F32) + bv[...]
    i_ids = lax.broadcasted_iota(jnp.int32, (l, jt), 0)
    j_ids = pl.program_id(1) * jt + lax.broadcasted_iota(jnp.int32, (l, jt), 1)
    ones_c = jnp.ones((C, 1), jnp.bfloat16)
    diag3 = (lax.broadcasted_iota(jnp.int32, (jt, jt, C), 0)
             == lax.broadcasted_iota(jnp.int32, (jt, jt, C), 1))
    ee_flat = ee.reshape(l, jt * C)
    als = []
    for h in range(H):
        sl = slice(h * C, (h + 1) * C)
        qh = q[:, sl]
        kh = k[:, sl]
        weh = we[:, sl]
        qk = lax.dot_general(kh, qh, (((1,), (1,)), ((), ())),
                             preferred_element_type=F32)
        qe = lax.dot_general(qh, weh, (((1,), (1,)), ((), ())),
                             preferred_element_type=F32)
        p_full = ee * qe.astype(jnp.bfloat16)[None]
        ae_log = jnp.dot(p_full.reshape(l * jt, C), ones_c,
                         preferred_element_type=F32).reshape(l, jt)
        logit = (qk + ae_log) * (1.0 / np.sqrt(C))
        logit = jnp.where(i_ids == j_ids, -1e30, logit)
        m = jnp.max(logit, axis=0, keepdims=True)
        ex = jnp.exp(logit - m)
        den = jnp.sum(ex, axis=0, keepdims=True)
        als.append(ex / (den + 1e-16))
    outs = []
    for h in range(H):
        sl = slice(h * C, (h + 1) * C)
        vh = v[:, sl]
        weh = we[:, sl]
        al = als[h]
        outv = lax.dot_general(al, vh, (((0,), (0,)), ((), ())),
                               preferred_element_type=F32)
        full = lax.dot_general(al.astype(jnp.bfloat16), ee_flat,
                               (((0,), (0,)), ((), ())),
                               preferred_element_type=F32)
        ae = jnp.sum(jnp.where(diag3, full.reshape(jt, jt, C), 0.0),
                     axis=0)
        oute = jnp.dot(ae, weh, preferred_element_type=F32)
        outs.append(outv + oute)
    out = jnp.concatenate(outs, axis=1)
    out = out + jnp.dot(gxj, wskip[...], preferred_element_type=F32) + bskip[...]
    out = _ln2d(out, lng[...], lnb[...])
    z = jnp.dot(out, wlin[...], preferred_element_type=F32) + blin[...] + gxj
    out_ref[0] = jnp.where(z > 0, z, jnp.exp(z) - 1.0)


def _unimp_block(gx, edge_e, bp):
    b, l = gx.shape[0], gx.shape[1]
    jt = 128
    args = (bp['q']['W'], bp['q']['b'].reshape(1, -1),
            bp['k']['W'], bp['k']['b'].reshape(1, -1),
            bp['v']['W'], bp['v']['b'].reshape(1, -1),
            bp['e']['W'],
            bp['skip']['W'], bp['skip']['b'].reshape(1, -1),
            bp['ln']['g'].reshape(1, -1), bp['ln']['be'].reshape(1, -1),
            bp['lin']['W'], bp['lin']['b'].reshape(1, -1))
    full = lambda a: pl.BlockSpec(a.shape, lambda bi, ji: tuple(0 for _ in a.shape))
    return pl.pallas_call(
        _block_body,
        grid=(b, l // jt),
        out_shape=jax.ShapeDtypeStruct((b, l, C), F32),
        in_specs=[pl.BlockSpec((1, l, C), lambda bi, ji: (bi, 0, 0)),
                  pl.BlockSpec((1, jt, C), lambda bi, ji: (bi, ji, 0)),
                  pl.BlockSpec((1, l, jt, C), lambda bi, ji: (bi, 0, ji, 0))]
                 + [full(a) for a in args],
        out_specs=pl.BlockSpec((1, jt, C), lambda bi, ji: (bi, ji, 0)),
    )(gx, gx, edge_e, *args)


def _sc_segsum(x_tab, src2, dst2, zeros_tab, n_iter, kch):
    r = x_tab.shape[0]
    rpt = r // 16
    n_rows = n_iter * kch
    n_grp = n_rows // (2 * kch)
    mesh = plsc.VectorSubcoreMesh(core_axis_name="c", subcore_axis_name="s")

    @functools.partial(
        pl.kernel,
        out_type=jax.ShapeDtypeStruct((2 * r, 64), F32),
        mesh=mesh,
        compiler_params=pltpu.CompilerParams(use_tc_tiling_on_sc=False),
        scratch_types=[
            pltpu.VMEM((n_rows, 128), jnp.int32),
            pltpu.VMEM((n_rows, 128), jnp.int32),
            pltpu.VMEM((2, kch, 128, 64), F32),
            pltpu.VMEM_SHARED((r, 64), F32),
            pltpu.SemaphoreType.DMA,
            pltpu.SemaphoreType.DMA,
        ],
    )
    def seg(x_hbm, src_hbm, dst_hbm, zero_hbm, out_hbm,
            srcv, dstv, rows, aggsh, gsem, ssem):
        ci = lax.axis_index("c")
        si = lax.axis_index("s")
        wid = ci * 16 + si
        pltpu.sync_copy(src_hbm.at[pl.ds(wid * n_rows, n_rows)], srcv)
        pltpu.sync_copy(dst_hbm.at[pl.ds(wid * n_rows, n_rows)], dstv)
        pltpu.sync_copy(zero_hbm.at[pl.ds(si * rpt, rpt)],
                        aggsh.at[pl.ds(si * rpt, rpt)])
        plsc.subcore_barrier()

        def body(gg, carry):
            base = gg * 2 * kch
            gd = []
            for par in (0, 1):
                gd.append([pltpu.async_copy(
                    x_hbm.at[srcv.at[base + par * kch + j]],
                    rows.at[par, j], gsem) for j in range(kch)])
            sd = []
            for par in (0, 1):
                for d in gd[par]:
                    d.wait()
                sd += [pltpu.async_copy(
                    rows.at[par, j], aggsh.at[dstv.at[base + par * kch + j]],
                    ssem, add=True) for j in range(kch)]
            for d in sd:
                d.wait()
            return carry

        lax.fori_loop(0, n_grp, body, 0)
        plsc.subcore_barrier()
        pltpu.sync_copy(aggsh.at[pl.ds(si * rpt, rpt)],
                        out_hbm.at[pl.ds(ci * r + si * rpt, rpt)])

    return seg(x_tab, src2, dst2, zeros_tab)


def _atom_lin_body(aggA, aggB, xin, wrel, brel, wroot, a, out_ref):
    agg = aggA[...] + aggB[...]
    y = jnp.dot(agg, wrel[...], preferred_element_type=F32) + brel[...]
    y = y + jnp.dot(xin[...], wroot[...], preferred_element_type=F32)
    out_ref[...] = jnp.where(y >= 0, y, a[...] * y)


def _atom_lin(aggA, aggB, xin, wrel, brel, wroot, a):
    r = xin.shape[0]
    rt = 2048
    full = lambda arr: pl.BlockSpec(arr.shape, lambda i: tuple(0 for _ in arr.shape))
    args = (wrel, brel, wroot, a)
    return pl.pallas_call(
        _atom_lin_body,
        grid=(r // rt,),
        out_shape=jax.ShapeDtypeStruct((r, 64), F32),
        in_specs=[pl.BlockSpec((rt, 64), lambda i: (i, 0))] * 3
                 + [full(arr) for arr in args],
        out_specs=pl.BlockSpec((rt, 64), lambda i: (i, 0)),
    )(aggA, aggB, xin, *args)


def _proj_body(x1, x2, x3, wp, bp_, nref, out_ref):
    rt = x1.shape[0]
    s = x1[...] + x2[...] + x3[...]
    y = jnp.dot(s, wp[...], preferred_element_type=F32) + bp_[...]
    glob = pl.program_id(0) * rt + lax.broadcasted_iota(jnp.int32, (rt, 64), 0)
    out_ref[...] = jnp.where(glob < nref[...], y, 0.0)


def _proj_atom(x1, x2, x3, wp, bp_, n_atom):
    r = x1.shape[0]
    rt = 2048
    nref = jnp.full((1, 1), n_atom, jnp.int32)
    full = lambda arr: pl.BlockSpec(arr.shape, lambda i: tuple(0 for _ in arr.shape))
    return pl.pallas_call(
        _proj_body,
        grid=(r // rt,),
        out_shape=jax.ShapeDtypeStruct((r, 64), F32),
        in_specs=[pl.BlockSpec((rt, 64), lambda i: (i, 0))] * 3
                 + [full(wp), full(bp_), full(nref)],
        out_specs=pl.BlockSpec((rt, 64), lambda i: (i, 0)),
    )(x1, x2, x3, wp, bp_, nref)


def _route_body(gx_ref, x3f_ref, u_ref, out_ref):
    gx = gx_ref[...]
    x3f = x3f_ref[...]
    gn = gx / (jnp.sqrt(jnp.sum(gx * gx, axis=1, keepdims=True)) + 1e-12)
    xn = x3f / (jnp.sqrt(jnp.sum(x3f * x3f, axis=1, keepdims=True)) + 1e-12)
    cos = lax.dot_general(gn, xn, (((1,), (1,)), ((), ())),
                          preferred_element_type=F32)
    m = jnp.max(cos, axis=0, keepdims=True)
    ex = jnp.exp(cos - m)
    den = jnp.sum(ex, axis=0, keepdims=True)
    sm = ex / den
    sm = jnp.where(u_ref[...] < sm, sm, 0.0)
    contrib = jnp.dot(sm, x3f, preferred_element_type=F32)

    @pl.when(pl.program_id(0) == 0)
    def _():
        out_ref[...] = contrib

    @pl.when(pl.program_id(0) != 0)
    def _():
        out_ref[...] = out_ref[...] + contrib


def _route(gx_flat, x3f, u_pad):
    n = gx_flat.shape[0]
    r = x3f.shape[0]
    ct = 1024
    return pl.pallas_call(
        _route_body,
        grid=(r // ct,),
        out_shape=jax.ShapeDtypeStruct((n, 64), F32),
        in_specs=[pl.BlockSpec((n, 64), lambda i: (0, 0)),
                  pl.BlockSpec((ct, 64), lambda i: (i, 0)),
                  pl.BlockSpec((n, ct), lambda i: (0, i))],
        out_specs=pl.BlockSpec((n, 64), lambda i: (0, 0)),
    )(gx_flat, x3f, u_pad)


def _enc_body(gx_ref, nx_ref, w1, w2, b0, out_ref):
    y = jnp.dot(gx_ref[...], w1[...], preferred_element_type=F32)
    y = y + jnp.dot(nx_ref[...], w2[...], preferred_element_type=F32) + b0[...]
    out_ref[...] = y


def _enc(gx_flat, new_x3, p):
    n = gx_flat.shape[0]
    w = p['res_atom_encoder']['W']
    args = (w[:C], w[C:], p['res_atom_encoder']['b'].reshape(1, -1))
    full = lambda arr: pl.BlockSpec(arr.shape, lambda: tuple(0 for _ in arr.shape))
    return pl.pallas_call(
        _enc_body,
        out_shape=jax.ShapeDtypeStruct((n, C), F32),
        in_specs=[full(gx_flat), full(new_x3)] + [full(a) for a in args],
        out_specs=pl.BlockSpec((n, C), lambda: (0, 0)),
    )(gx_flat, new_x3, *args)


def _heads_body(gx_ref, wx, bx, sg, sb, wst, bst, xyz_ref, st_ref):
    gx = gx_ref[...]
    xyz_ref[...] = jnp.dot(gx, wx[...], preferred_element_type=F32) + bx[...]
    gn = _ln(gx, sg[...], sb[...])
    st_ref[...] = jnp.dot(gn, wst[...], preferred_element_type=F32) + bst[...]


def _heads(gx_flat, p):
    n = gx_flat.shape[0]
    args = (p['get_xyz']['W'], p['get_xyz']['b'].reshape(1, -1),
            p['norm_state']['g'].reshape(1, -1), p['norm_state']['be'].reshape(1, -1),
            p['get_state']['W'], p['get_state']['b'].reshape(1, -1))
    d_xyz = p['get_xyz']['W'].shape[1]
    d_st = p['get_state']['W'].shape[1]
    full = lambda arr: pl.BlockSpec(arr.shape, lambda: tuple(0 for _ in arr.shape))
    return pl.pallas_call(
        _heads_body,
        out_shape=(jax.ShapeDtypeStruct((n, d_xyz), F32),
                   jax.ShapeDtypeStruct((n, d_st), F32)),
        in_specs=[full(gx_flat)] + [full(a) for a in args],
        out_specs=(pl.BlockSpec((n, d_xyz), lambda: (0, 0)),
                   pl.BlockSpec((n, d_st), lambda: (0, 0))),
    )(gx_flat, *args)


def kernel(seq1hot, idx, node, edge, x, edge_index, params):
    p = params
    bc, lc = node.shape[0], node.shape[1]
    n = bc * lc

    gx0 = _node_embed(node.reshape(n, -1), seq1hot.reshape(n, -1), p)
    edge_e = _edge_embed(edge, p)
    gx = gx0.reshape(bc, lc, C)
    for bp in p['blocks']:
        gx = _unimp_block(gx, edge_e, bp)

    n_atom = x.shape[0]
    rt = 2048
    r = ((n_atom + rt - 1) // rt) * rt
    x64 = jnp.zeros((r, 64), F32).at[:n_atom, :x.shape[1]].set(x)
    src, dst = edge_index[0], edge_index[1]
    e_num = src.shape[0]
    kch = 4
    n_rows_t = -(-e_num // (32 * 128))
    n_rows_t = -(-n_rows_t // (2 * kch)) * (2 * kch)
    n_iter = n_rows_t // kch
    e_pad = 32 * n_rows_t * 128
    src_p = jnp.concatenate([src, jnp.zeros((e_pad - e_num,), jnp.int32)])
    dst_p = jnp.concatenate([dst, jnp.full((e_pad - e_num,), r - 1, jnp.int32)])
    src2 = src_p.reshape(-1, 128)
    dst2 = dst_p.reshape(-1, 128)
    zeros_tab = jnp.zeros((r, 64), F32)

    def pad_w(w):
        return jnp.zeros((64, 64), F32).at[:w.shape[0]].set(w)

    a = p['prelu_a'].reshape(1, 1)
    agg = _sc_segsum(x64, src2, dst2, zeros_tab, n_iter, kch)
    x1 = _atom_lin(agg[:r], agg[r:], x64, pad_w(p['conv1_rel']['W']),
                   p['conv1_rel']['b'].reshape(1, -1), pad_w(p['conv1_root']['W']), a)
    agg = _sc_segsum(x1, src2, dst2, zeros_tab, n_iter, kch)
    x2 = _atom_lin(agg[:r], agg[r:], x1, p['conv2_rel']['W'],
                   p['conv2_rel']['b'].reshape(1, -1), p['conv2_root']['W'], a)
    agg = _sc_segsum(x2, src2, dst2, zeros_tab, n_iter, kch)
    x3 = _atom_lin(agg[:r], agg[r:], x2, p['conv3_rel']['W'],
                   p['conv3_rel']['b'].reshape(1, -1), p['conv3_root']['W'], a)
    x3f = _proj_atom(x1, x2, x3, p['proj_atom']['W'],
                     p['proj_atom']['b'].reshape(1, -1), n_atom)

    u = jax.random.uniform(jax.random.key(42), (n, n_atom), F32)
    u_pad = jnp.ones((n, r), F32).at[:, :n_atom].set(u)
    gx_flat = gx.reshape(n, C)
    new_x3 = _route(gx_flat, x3f, u_pad)

    gx2 = _enc(gx_flat, new_x3, p)
    gx3 = _unimp_block(gx2.reshape(bc, lc, C), edge_e, p['final_block'])
    xyz, state = _heads(gx3.reshape(n, C), p)
    return xyz.reshape(bc, lc, 3, 3), state.reshape(bc, lc, -1)
